# Initial kernel scaffold; baseline (speedup 1.0000x reference)
import jax
import jax.numpy as jnp
from jax.experimental import pallas as pl


def kernel(data, emb0, emb1):
    raise NotImplementedError("write your pallas kernel here")


# speedup vs baseline: 8.0550x; 8.0550x over previous
import numpy as np
import jax
import jax.numpy as jnp
from jax import lax
from jax.experimental import pallas as pl
from jax.experimental.pallas import tpu as pltpu
from jax.experimental.pallas import tpu_sc as plsc

_VOCAB = 1000000
_DIM = 128
_NEG = 10
_DROP = 0.1
_B = 16384

_NC = 2
_NS = 16
_NW = _NC * _NS
_BPW = _B // _NW
_G = 128
_WG = _BPW // _G
_NGRP = _BPW * _NEG // _G

def _dropout_mask():
    mk = jax.random.key(12345)
    return (jax.random.uniform(mk, (_B, _DIM)) < (1.0 - _DROP)).astype(jnp.float32)


def _sc_gather_body(emb0, emb1, widx, cidx, nidx, w_out, c_out, n_out,
                    idxw_v, idxc_v, idxn_v, buf0, sem0):
    wid = lax.axis_index("s") * _NC + lax.axis_index("c")
    pltpu.sync_copy(widx.at[pl.ds(wid * _WG, _WG)], idxw_v)
    pltpu.sync_copy(cidx.at[pl.ds(wid * _WG, _WG)], idxc_v)
    pltpu.sync_copy(nidx.at[pl.ds(wid * _NGRP, _NGRP)], idxn_v)
    for j in range(_WG):
        pltpu.async_copy(emb0.at[idxw_v.at[j]], buf0, sem0).wait()
        pltpu.sync_copy(buf0, w_out.at[pl.ds(wid * _BPW + j * _G, _G)])
    for j in range(_WG):
        pltpu.async_copy(emb1.at[idxc_v.at[j]], buf0, sem0).wait()
        pltpu.sync_copy(buf0, c_out.at[pl.ds(wid * _BPW + j * _G, _G)])

    def neg_step(j, carry):
        pltpu.async_copy(emb1.at[idxn_v.at[j]], buf0, sem0).wait()
        pltpu.sync_copy(buf0, n_out.at[pl.ds(wid * _BPW * _NEG + j * _G, _G)])
        return carry

    lax.fori_loop(0, _NGRP, neg_step, 0)


_sc_gather = pl.kernel(
    _sc_gather_body,
    out_type=[
        jax.ShapeDtypeStruct((_B, _DIM), jnp.float32),
        jax.ShapeDtypeStruct((_B, _DIM), jnp.float32),
        jax.ShapeDtypeStruct((_B * _NEG, _DIM), jnp.float32),
    ],
    mesh=plsc.VectorSubcoreMesh(core_axis_name="c", subcore_axis_name="s"),
    scratch_types=[
        pltpu.VMEM((_WG, _G), jnp.int32),
        pltpu.VMEM((_WG, _G), jnp.int32),
        pltpu.VMEM((_NGRP, _G), jnp.int32),
        pltpu.VMEM((_G, _DIM), jnp.float32),
        pltpu.SemaphoreType.DMA,
    ],
)

_R = 512
_GRID = _B // _R


def _loss_body(w_ref, c_ref, m_ref, n_ref, nm_ref, out_ref):
    i = pl.program_id(0)
    wm = w_ref[...] * m_ref[...]
    pos = jnp.sum(wm * c_ref[...], axis=1, keepdims=True)
    pos_l = jnp.sum(jnp.log(1.0 + jnp.exp(-jnp.clip(pos, -10.0, 10.0))))
    n3 = n_ref[...]
    negs = jnp.sum(n3 * wm[:, None, :], axis=2, keepdims=True)
    neg_l = jnp.sum(
        jnp.log(1.0 + jnp.exp(jnp.clip(negs, -10.0, 10.0))) * nm_ref[...])

    @pl.when(i == 0)
    def _():
        out_ref[...] = jnp.zeros_like(out_ref)

    out_ref[...] += jnp.reshape(pos_l + neg_l, (1, 1))


_loss_call = pl.pallas_call(
    _loss_body,
    grid=(_GRID,),
    in_specs=[
        pl.BlockSpec((_R, _DIM), lambda i: (i, 0)),
        pl.BlockSpec((_R, _DIM), lambda i: (i, 0)),
        pl.BlockSpec((_R, _DIM), lambda i: (i, 0)),
        pl.BlockSpec((_R, _NEG, _DIM), lambda i: (i, 0, 0)),
        pl.BlockSpec((_R, _NEG, 1), lambda i: (i, 0, 0)),
    ],
    out_specs=pl.BlockSpec((1, 1), lambda i: (0, 0)),
    out_shape=jax.ShapeDtypeStruct((1, 1), jnp.float32),
)


def kernel(data, emb0, emb1):
    word_idx = data[:, 1].reshape(_B // _G, _G)
    ctx_idx = data[:, 0].reshape(_B // _G, _G)
    neg_idx = data[:, 2:2 + _NEG].reshape(_B * _NEG // _G, _G)
    neg_mask = data[:, 2 + _NEG:].astype(jnp.float32).reshape(_B, _NEG, 1)
    mask = _dropout_mask()
    w, c, n = _sc_gather(emb0, emb1, word_idx, ctx_idx, neg_idx)
    out = _loss_call(w, c, mask, n.reshape(_B, _NEG, _DIM), neg_mask)
    return out[0, 0]


# speedup vs baseline: 23.5247x; 2.9205x over previous
import jax
import jax.numpy as jnp
from jax import lax
from jax.experimental import pallas as pl
from jax.experimental.pallas import tpu as pltpu
from jax.experimental.pallas import tpu_sc as plsc

_VOCAB = 1000000
_DIM = 128
_NEG = 10
_DROP = 0.1
_B = 16384

_NC = 2
_NS = 16
_NW = _NC * _NS
_BPW = _B // _NW
_C = 32
_CH = _BPW // _C
_NPC = _C * _NEG
_NSTREAM = 5
_NROWS = _NPC // _NSTREAM


def _dropout_mask():
    mk = jax.random.key(12345)
    return (jax.random.uniform(mk, (_B, _DIM)) < (1.0 - _DROP)).astype(jnp.float32)


def _sc_body(emb0, emb1, widx, cidx, nidx, maskh, pos_out, neg_out,
             idxw, idxc, idxn, wbuf0, wbuf1, cbuf0, cbuf1, mbuf0, mbuf1,
             nbuf0, nbuf1, posbuf, negbuf, sem0, sem1):
    wid = lax.axis_index("s") * _NC + lax.axis_index("c")
    base = wid * _BPW
    pltpu.sync_copy(widx.at[pl.ds(wid * _CH, _CH)], idxw)
    pltpu.sync_copy(cidx.at[pl.ds(wid * _CH, _CH)], idxc)
    pltpu.sync_copy(nidx.at[pl.ds(wid * _CH * _NSTREAM, _CH * _NSTREAM)],
                    idxn)
    lanes = lax.iota(jnp.int32, 16)

    bufs = ((wbuf0, cbuf0, mbuf0, nbuf0, sem0),
            (wbuf1, cbuf1, mbuf1, nbuf1, sem1))

    def fire(ch):
        wb, cb, mb, nb, sem = bufs[ch % 2]
        cps = [
            pltpu.async_copy(emb0.at[idxw.at[ch]], wb, sem),
            pltpu.async_copy(emb1.at[idxc.at[ch]], cb, sem),
            pltpu.async_copy(maskh.at[pl.ds(base + ch * _C, _C)], mb, sem),
        ]
        for j in range(_NSTREAM):
            cps.append(pltpu.async_copy(
                emb1.at[idxn.at[ch * _NSTREAM + j]],
                nb.at[pl.ds(j * _NROWS, _NROWS)], sem))
        return cps

    def compute(ch):
        wb, cb, mb, nb, _ = bufs[ch % 2]
        off = ch * _C

        def rowstep(r, carry):
            wmv = [wb[r, pl.ds(v * 16, 16)] * mb[r, pl.ds(v * 16, 16)]
                   for v in range(_DIM // 16)]
            lanemask = lanes == (r & 15)
            p = wmv[0] * cb[r, pl.ds(0, 16)]
            for v in range(1, _DIM // 16):
                p = p + wmv[v] * cb[r, pl.ds(v * 16, 16)]
            accs = [jnp.where(lanemask, lax.reduce_sum(p, (0,)), carry[0])]
            for k in range(_NEG):
                nr = r * _NEG + k
                q = wmv[0] * nb[nr, pl.ds(0, 16)]
                for v in range(1, _DIM // 16):
                    q = q + wmv[v] * nb[nr, pl.ds(v * 16, 16)]
                accs.append(
                    jnp.where(lanemask, lax.reduce_sum(q, (0,)), carry[1 + k]))
            flush = (r & 15) == 15

            @pl.when(flush)
            def _():
                posbuf[pl.ds(off + r - 15, 16)] = accs[0]
                for k in range(_NEG):
                    negbuf[k, pl.ds(off + r - 15, 16)] = accs[1 + k]

            return tuple(jnp.where(flush, 0.0, a) for a in accs)

        zeros = tuple(jnp.zeros((16,), jnp.float32) for _ in range(_NEG + 1))
        lax.fori_loop(0, _C, rowstep, zeros)

    pending = fire(0)
    for ch in range(_CH):
        for cp in pending:
            cp.wait()
        if ch + 1 < _CH:
            pending = fire(ch + 1)
        compute(ch)

    pltpu.sync_copy(posbuf, pos_out.at[pl.ds(base, _BPW)])
    pltpu.sync_copy(negbuf, neg_out.at[wid])


_sc_call = pl.kernel(
    _sc_body,
    out_type=[
        jax.ShapeDtypeStruct((_B,), jnp.float32),
        jax.ShapeDtypeStruct((_NW, _NEG, _BPW), jnp.float32),
    ],
    mesh=plsc.VectorSubcoreMesh(core_axis_name="c", subcore_axis_name="s"),
    compiler_params=pltpu.CompilerParams(needs_layout_passes=False),
    scratch_types=[
        pltpu.VMEM((_CH, _C), jnp.int32),
        pltpu.VMEM((_CH, _C), jnp.int32),
        pltpu.VMEM((_CH * _NSTREAM, _NROWS), jnp.int32),
        pltpu.VMEM((_C, _DIM), jnp.float32),
        pltpu.VMEM((_C, _DIM), jnp.float32),
        pltpu.VMEM((_C, _DIM), jnp.float32),
        pltpu.VMEM((_C, _DIM), jnp.float32),
        pltpu.VMEM((_C, _DIM), jnp.float32),
        pltpu.VMEM((_C, _DIM), jnp.float32),
        pltpu.VMEM((_NPC, _DIM), jnp.float32),
        pltpu.VMEM((_NPC, _DIM), jnp.float32),
        pltpu.VMEM((_BPW,), jnp.float32),
        pltpu.VMEM((_NEG, _BPW), jnp.float32),
        pltpu.SemaphoreType.DMA,
        pltpu.SemaphoreType.DMA,
    ],
)


def _loss_body(pos_ref, negs_ref, nm_ref, out_ref):
    pos = pos_ref[...]
    pos_l = jnp.sum(jnp.log(1.0 + jnp.exp(-jnp.clip(pos, -10.0, 10.0))))
    negs = negs_ref[...]
    nl = jnp.log(1.0 + jnp.exp(jnp.clip(negs, -10.0, 10.0))) * nm_ref[...]
    out_ref[...] = jnp.reshape(pos_l + jnp.sum(nl), (1, 1))


_loss_call = pl.pallas_call(
    _loss_body,
    out_shape=jax.ShapeDtypeStruct((1, 1), jnp.float32),
)


def kernel(data, emb0, emb1):
    word_idx = data[:, 1].reshape(_B // _C, _C)
    ctx_idx = data[:, 0].reshape(_B // _C, _C)
    neg_idx = data[:, 2:2 + _NEG].reshape(_B * _NEG // _NROWS, _NROWS)
    nm = (data[:, 2 + _NEG:].astype(jnp.float32)
          .reshape(_NW, _BPW, _NEG).transpose(0, 2, 1))
    mask = _dropout_mask()
    pos, negs = _sc_call(emb0, emb1, word_idx, ctx_idx, neg_idx, mask)
    out = _loss_call(pos.reshape(_B // _DIM, _DIM), negs, nm)
    return out[0, 0]


# speedup vs baseline: 24.4303x; 1.0385x over previous
import jax
import jax.numpy as jnp
from jax import lax
from jax.experimental import pallas as pl
from jax.experimental.pallas import tpu as pltpu
from jax.experimental.pallas import tpu_sc as plsc

_VOCAB = 1000000
_DIM = 128
_NEG = 10
_DROP = 0.1
_B = 16384

_NC = 2
_NS = 16
_NW = _NC * _NS
_BPW = _B // _NW
_C = 16
_CH = _BPW // _C
_NPC = _C * _NEG
_HALF = _BPW // 4


def _dropout_mask():
    mk = jax.random.key(12345)
    return (jax.random.uniform(mk, (_B, _DIM)) < (1.0 - _DROP)).astype(jnp.float32)


def _sc_body(data, emb0, emb1, maskh, pos_out, neg_out, nmask_out,
             dbuf, idxw, idxc, idxn, wbuf0, wbuf1, cbuf0, cbuf1,
             mbuf0, mbuf1, nbuf0, nbuf1, posbuf, negbuf, nmaskbuf,
             sem0, sem1):
    wid = lax.axis_index("s") * _NC + lax.axis_index("c")
    base = wid * _BPW
    lanes = lax.iota(jnp.int32, 16)

    for h in range(_BPW // _HALF):
        pltpu.sync_copy(data.at[pl.ds(base + h * _HALF, _HALF)], dbuf)

        def extract(g, carry):
            rows = lanes + g * 16
            out0 = h * _HALF + g * 16
            wv = plsc.load_gather(dbuf, [rows, jnp.full((16,), 1, jnp.int32)])
            cv = plsc.load_gather(dbuf, [rows, jnp.full((16,), 0, jnp.int32)])
            idxw[pl.ds(out0, 16)] = wv
            idxc[pl.ds(out0, 16)] = cv
            for k in range(_NEG):
                nv = plsc.load_gather(
                    dbuf, [rows, jnp.full((16,), 2 + k, jnp.int32)])
                idxn[k, pl.ds(out0, 16)] = nv
                mv = plsc.load_gather(
                    dbuf, [rows, jnp.full((16,), 2 + _NEG + k, jnp.int32)])
                nmaskbuf[k, pl.ds(out0, 16)] = mv.astype(jnp.float32)
            return carry

        lax.fori_loop(0, _HALF // 16, extract, 0)

    bufs = ((wbuf0, cbuf0, mbuf0, nbuf0, sem0),
            (wbuf1, cbuf1, mbuf1, nbuf1, sem1))

    def _copies(ch, p):
        wb, cb, mb, nb, sem = bufs[p]
        yield emb0.at[idxw.at[pl.ds(ch * _C, _C)]], wb, sem
        yield emb1.at[idxc.at[pl.ds(ch * _C, _C)]], cb, sem
        yield maskh.at[pl.ds(base + ch * _C, _C)], mb, sem
        for k in range(_NEG):
            yield (emb1.at[idxn.at[k, pl.ds(ch * _C, _C)]],
                   nb.at[pl.ds(k * _C, _C)], sem)

    def fire(ch, p):
        for src, dst, sem in _copies(ch, p):
            pltpu.async_copy(src, dst, sem)

    def drain(ch, p):
        for src, dst, sem in _copies(ch, p):
            pltpu.make_async_copy(src, dst, sem).wait()

    def compute(ch, p):
        wb, cb, mb, nb, _ = bufs[p]
        off = ch * _C

        def rowstep(r, carry):
            wmv = [wb[r, pl.ds(v * 16, 16)] * mb[r, pl.ds(v * 16, 16)]
                   for v in range(_DIM // 16)]
            lanemask = lanes == (r & 15)
            p = wmv[0] * cb[r, pl.ds(0, 16)]
            for v in range(1, _DIM // 16):
                p = p + wmv[v] * cb[r, pl.ds(v * 16, 16)]
            accs = [jnp.where(lanemask, lax.reduce_sum(p, (0,)), carry[0])]
            for k in range(_NEG):
                nr = k * _C + r
                q = wmv[0] * nb[nr, pl.ds(0, 16)]
                for v in range(1, _DIM // 16):
                    q = q + wmv[v] * nb[nr, pl.ds(v * 16, 16)]
                accs.append(
                    jnp.where(lanemask, lax.reduce_sum(q, (0,)), carry[1 + k]))
            flush = (r & 15) == 15

            @pl.when(flush)
            def _():
                posbuf[pl.ds(off + r - 15, 16)] = accs[0]
                for k in range(_NEG):
                    negbuf[k, pl.ds(off + r - 15, 16)] = accs[1 + k]

            return tuple(jnp.where(flush, 0.0, a) for a in accs)

        zeros = tuple(jnp.zeros((16,), jnp.float32) for _ in range(_NEG + 1))
        lax.fori_loop(0, _C, rowstep, zeros)

    fire(0, 0)
    fire(1, 1)

    def pair(j, carry):
        ch0 = j * 2
        drain(ch0, 0)
        compute(ch0, 0)

        @pl.when(ch0 + 2 < _CH)
        def _():
            fire(ch0 + 2, 0)

        drain(ch0 + 1, 1)
        compute(ch0 + 1, 1)

        @pl.when(ch0 + 3 < _CH)
        def _():
            fire(ch0 + 3, 1)

        return carry

    lax.fori_loop(0, _CH // 2, pair, 0)

    pltpu.sync_copy(posbuf, pos_out.at[pl.ds(base, _BPW)])
    pltpu.sync_copy(negbuf, neg_out.at[wid])
    pltpu.sync_copy(nmaskbuf, nmask_out.at[wid])


_sc_call = pl.kernel(
    _sc_body,
    out_type=[
        jax.ShapeDtypeStruct((_B,), jnp.float32),
        jax.ShapeDtypeStruct((_NW, _NEG, _BPW), jnp.float32),
        jax.ShapeDtypeStruct((_NW, _NEG, _BPW), jnp.float32),
    ],
    mesh=plsc.VectorSubcoreMesh(core_axis_name="c", subcore_axis_name="s"),
    compiler_params=pltpu.CompilerParams(needs_layout_passes=False),
    scratch_types=[
        pltpu.VMEM((_HALF, 2 + 2 * _NEG), jnp.int32),
        pltpu.VMEM((_BPW,), jnp.int32),
        pltpu.VMEM((_BPW,), jnp.int32),
        pltpu.VMEM((_NEG, _BPW), jnp.int32),
        pltpu.VMEM((_C, _DIM), jnp.float32),
        pltpu.VMEM((_C, _DIM), jnp.float32),
        pltpu.VMEM((_C, _DIM), jnp.float32),
        pltpu.VMEM((_C, _DIM), jnp.float32),
        pltpu.VMEM((_C, _DIM), jnp.float32),
        pltpu.VMEM((_C, _DIM), jnp.float32),
        pltpu.VMEM((_NPC, _DIM), jnp.float32),
        pltpu.VMEM((_NPC, _DIM), jnp.float32),
        pltpu.VMEM((_BPW,), jnp.float32),
        pltpu.VMEM((_NEG, _BPW), jnp.float32),
        pltpu.VMEM((_NEG, _BPW), jnp.float32),
        pltpu.SemaphoreType.DMA,
        pltpu.SemaphoreType.DMA,
    ],
)


def _loss_body(pos_ref, negs_ref, nm_ref, out_ref):
    pos = pos_ref[...]
    pos_l = jnp.sum(jnp.log(1.0 + jnp.exp(-jnp.clip(pos, -10.0, 10.0))))
    negs = negs_ref[...]
    nl = jnp.log(1.0 + jnp.exp(jnp.clip(negs, -10.0, 10.0))) * nm_ref[...]
    out_ref[...] = jnp.reshape(pos_l + jnp.sum(nl), (1, 1))


_loss_call = pl.pallas_call(
    _loss_body,
    out_shape=jax.ShapeDtypeStruct((1, 1), jnp.float32),
)


def kernel(data, emb0, emb1):
    mask = _dropout_mask()
    pos, negs, nmask = _sc_call(data, emb0, emb1, mask)
    out = _loss_call(pos.reshape(_B // _DIM, _DIM), negs, nmask)
    return out[0, 0]


# speedup vs baseline: 31.7352x; 1.2990x over previous
import jax
import jax.numpy as jnp
from jax import lax
from jax.experimental import pallas as pl
from jax.experimental.pallas import tpu as pltpu
from jax.experimental.pallas import tpu_sc as plsc

_VOCAB = 1000000
_DIM = 128
_NEG = 10
_DROP = 0.1
_B = 16384

_NC = 2
_NS = 16
_NW = _NC * _NS
_BPW = _B // _NW
_C = 16
_CH = _BPW // _C
_NPC = _C * _NEG
_HALF = _BPW // 4


def _dropout_mask():
    with jax.ensure_compile_time_eval():
        mk = jax.random.key(12345)
        return (jax.random.uniform(mk, (_B, _DIM))
                < (1.0 - _DROP)).astype(jnp.float32)


def _sc_body(data, emb0, emb1, maskh, pos_out, neg_out, nmask_out,
             dbuf, idxw, idxc, idxn, wbuf0, wbuf1, cbuf0, cbuf1,
             mbuf0, mbuf1, nbuf0, nbuf1, posbuf, negbuf, nmaskbuf,
             sem0, sem1):
    wid = lax.axis_index("s") * _NC + lax.axis_index("c")
    base = wid * _BPW
    lanes = lax.iota(jnp.int32, 16)

    for h in range(_BPW // _HALF):
        pltpu.sync_copy(data.at[pl.ds(base + h * _HALF, _HALF)], dbuf)

        def extract(g, carry):
            rows = lanes + g * 16
            out0 = h * _HALF + g * 16
            wv = plsc.load_gather(dbuf, [rows, jnp.full((16,), 1, jnp.int32)])
            cv = plsc.load_gather(dbuf, [rows, jnp.full((16,), 0, jnp.int32)])
            idxw[pl.ds(out0, 16)] = wv
            idxc[pl.ds(out0, 16)] = cv
            for k in range(_NEG):
                nv = plsc.load_gather(
                    dbuf, [rows, jnp.full((16,), 2 + k, jnp.int32)])
                idxn[k, pl.ds(out0, 16)] = nv
                mv = plsc.load_gather(
                    dbuf, [rows, jnp.full((16,), 2 + _NEG + k, jnp.int32)])
                nmaskbuf[k, pl.ds(out0, 16)] = mv.astype(jnp.float32)
            return carry

        lax.fori_loop(0, _HALF // 16, extract, 0)

    bufs = ((wbuf0, cbuf0, mbuf0, nbuf0, sem0),
            (wbuf1, cbuf1, mbuf1, nbuf1, sem1))

    def _copies(ch, p):
        wb, cb, mb, nb, sem = bufs[p]
        yield emb0.at[idxw.at[pl.ds(ch * _C, _C)]], wb, sem
        yield emb1.at[idxc.at[pl.ds(ch * _C, _C)]], cb, sem
        yield maskh.at[pl.ds(base + ch * _C, _C)], mb, sem
        for k in range(_NEG):
            yield (emb1.at[idxn.at[k, pl.ds(ch * _C, _C)]],
                   nb.at[pl.ds(k * _C, _C)], sem)

    def fire(ch, p):
        for src, dst, sem in _copies(ch, p):
            pltpu.async_copy(src, dst, sem)

    def drain(ch, p):
        for src, dst, sem in _copies(ch, p):
            pltpu.make_async_copy(src, dst, sem).wait()

    def compute(ch, p):
        wb, cb, mb, nb, _ = bufs[p]
        off = ch * _C

        def rowstep(r, carry):
            wmv = [wb[r, pl.ds(v * 16, 16)] * mb[r, pl.ds(v * 16, 16)]
                   for v in range(_DIM // 16)]
            lanemask = lanes == (r & 15)
            p = wmv[0] * cb[r, pl.ds(0, 16)]
            for v in range(1, _DIM // 16):
                p = p + wmv[v] * cb[r, pl.ds(v * 16, 16)]
            accs = [jnp.where(lanemask, lax.reduce_sum(p, (0,)), carry[0])]
            for k in range(_NEG):
                nr = k * _C + r
                q = wmv[0] * nb[nr, pl.ds(0, 16)]
                for v in range(1, _DIM // 16):
                    q = q + wmv[v] * nb[nr, pl.ds(v * 16, 16)]
                accs.append(
                    jnp.where(lanemask, lax.reduce_sum(q, (0,)), carry[1 + k]))
            flush = (r & 15) == 15

            @pl.when(flush)
            def _():
                posbuf[pl.ds(off + r - 15, 16)] = accs[0]
                for k in range(_NEG):
                    negbuf[k, pl.ds(off + r - 15, 16)] = accs[1 + k]

            return tuple(jnp.where(flush, 0.0, a) for a in accs)

        zeros = tuple(jnp.zeros((16,), jnp.float32) for _ in range(_NEG + 1))
        lax.fori_loop(0, _C, rowstep, zeros)

    fire(0, 0)
    fire(1, 1)

    def pair(j, carry):
        ch0 = j * 2
        drain(ch0, 0)
        compute(ch0, 0)

        @pl.when(ch0 + 2 < _CH)
        def _():
            fire(ch0 + 2, 0)

        drain(ch0 + 1, 1)
        compute(ch0 + 1, 1)

        @pl.when(ch0 + 3 < _CH)
        def _():
            fire(ch0 + 3, 1)

        return carry

    lax.fori_loop(0, _CH // 2, pair, 0)

    pltpu.sync_copy(posbuf, pos_out.at[pl.ds(base, _BPW)])
    pltpu.sync_copy(negbuf, neg_out.at[wid])
    pltpu.sync_copy(nmaskbuf, nmask_out.at[wid])


_sc_call = pl.kernel(
    _sc_body,
    out_type=[
        jax.ShapeDtypeStruct((_B,), jnp.float32),
        jax.ShapeDtypeStruct((_NW, _NEG, _BPW), jnp.float32),
        jax.ShapeDtypeStruct((_NW, _NEG, _BPW), jnp.float32),
    ],
    mesh=plsc.VectorSubcoreMesh(core_axis_name="c", subcore_axis_name="s"),
    compiler_params=pltpu.CompilerParams(needs_layout_passes=False),
    scratch_types=[
        pltpu.VMEM((_HALF, 2 + 2 * _NEG), jnp.int32),
        pltpu.VMEM((_BPW,), jnp.int32),
        pltpu.VMEM((_BPW,), jnp.int32),
        pltpu.VMEM((_NEG, _BPW), jnp.int32),
        pltpu.VMEM((_C, _DIM), jnp.float32),
        pltpu.VMEM((_C, _DIM), jnp.float32),
        pltpu.VMEM((_C, _DIM), jnp.float32),
        pltpu.VMEM((_C, _DIM), jnp.float32),
        pltpu.VMEM((_C, _DIM), jnp.float32),
        pltpu.VMEM((_C, _DIM), jnp.float32),
        pltpu.VMEM((_NPC, _DIM), jnp.float32),
        pltpu.VMEM((_NPC, _DIM), jnp.float32),
        pltpu.VMEM((_BPW,), jnp.float32),
        pltpu.VMEM((_NEG, _BPW), jnp.float32),
        pltpu.VMEM((_NEG, _BPW), jnp.float32),
        pltpu.SemaphoreType.DMA,
        pltpu.SemaphoreType.DMA,
    ],
)


def _loss_body(pos_ref, negs_ref, nm_ref, out_ref):
    pos = pos_ref[...]
    pos_l = jnp.sum(jnp.log(1.0 + jnp.exp(-jnp.clip(pos, -10.0, 10.0))))
    negs = negs_ref[...]
    nl = jnp.log(1.0 + jnp.exp(jnp.clip(negs, -10.0, 10.0))) * nm_ref[...]
    out_ref[...] = jnp.reshape(pos_l + jnp.sum(nl), (1, 1))


_loss_call = pl.pallas_call(
    _loss_body,
    out_shape=jax.ShapeDtypeStruct((1, 1), jnp.float32),
)


def kernel(data, emb0, emb1):
    mask = _dropout_mask()
    pos, negs, nmask = _sc_call(data, emb0, emb1, mask)
    out = _loss_call(pos.reshape(_B // _DIM, _DIM), negs, nmask)
    return out[0, 0]


# speedup vs baseline: 37.3510x; 1.1770x over previous
import jax
import jax.numpy as jnp
from jax import lax
from jax.experimental import pallas as pl
from jax.experimental.pallas import tpu as pltpu
from jax.experimental.pallas import tpu_sc as plsc

_VOCAB = 1000000
_DIM = 128
_NEG = 10
_DROP = 0.1
_B = 16384

_NC = 2
_NS = 16
_NW = _NC * _NS
_BPW = _B // _NW
_C = 32
_CH = _BPW // _C
_NPC = _C * _NEG
_WPB = _DIM // 32


def _dropout_mask_bits():
    with jax.ensure_compile_time_eval():
        mk = jax.random.key(12345)
        bits = (jax.random.uniform(mk, (_B, _DIM)) < (1.0 - _DROP))
        b3 = bits.reshape(_B, _WPB, 32).astype(jnp.uint32)
        words = jnp.sum(b3 << jnp.arange(32, dtype=jnp.uint32)[None, None, :],
                        axis=2, dtype=jnp.uint32)
        return lax.bitcast_convert_type(words, jnp.int32).reshape(_B * _WPB)


def _sc_body(widx, cidx, nidx, emb0, emb1, mbits, pos_out, neg_out,
             idxwv, idxcv, idxnv, bitsv, wbuf0, wbuf1, cbuf0, cbuf1,
             nbuf0, nbuf1, posbuf, negbuf, sem0, sem1):
    wid = lax.axis_index("s") * _NC + lax.axis_index("c")
    base = wid * _BPW
    lanes = lax.iota(jnp.int32, 16)
    bitsel = (jnp.full((16,), 1, jnp.int32) << lanes,
              jnp.full((16,), 1, jnp.int32) << (lanes + 16))

    pltpu.sync_copy(widx.at[pl.ds(base, _BPW)], idxwv)
    pltpu.sync_copy(cidx.at[pl.ds(base, _BPW)], idxcv)
    for k in range(_NEG):
        pltpu.sync_copy(nidx.at[pl.ds(k * _B + base, _BPW)],
                        idxnv.at[pl.ds(k * _BPW, _BPW)])
    pltpu.sync_copy(mbits.at[pl.ds(base * _WPB, _BPW * _WPB)], bitsv)

    bufs = ((wbuf0, cbuf0, nbuf0, sem0), (wbuf1, cbuf1, nbuf1, sem1))

    def _copies(ch, p):
        wb, cb, nb, sem = bufs[p]
        yield emb0.at[idxwv.at[pl.ds(ch * _C, _C)]], wb, sem
        yield emb1.at[idxcv.at[pl.ds(ch * _C, _C)]], cb, sem
        for k in range(_NEG):
            yield (emb1.at[idxnv.at[pl.ds(k * _BPW + ch * _C, _C)]],
                   nb.at[pl.ds(k * _C, _C)], sem)

    def fire(ch, p):
        for src, dst, sem in _copies(ch, p):
            pltpu.async_copy(src, dst, sem)

    def drain(ch, p):
        for src, dst, sem in _copies(ch, p):
            pltpu.make_async_copy(src, dst, sem).wait()

    def compute(ch, p):
        wb, cb, nb, _ = bufs[p]
        off = ch * _C

        def rowstep(r, carry):
            wordbase = (off + r) * _WPB
            wmv = []
            for pair in range(_WPB):
                bw = plsc.load_gather(
                    bitsv, [jnp.full((16,), wordbase + pair, jnp.int32)])
                for j in range(2):
                    v = 2 * pair + j
                    wv = wb[r, pl.ds(v * 16, 16)]
                    wmv.append(jnp.where((bw & bitsel[j]) != 0, wv, 0.0))
            lanemask = lanes == (r & 15)
            p_ = wmv[0] * cb[r, pl.ds(0, 16)]
            for v in range(1, _DIM // 16):
                p_ = p_ + wmv[v] * cb[r, pl.ds(v * 16, 16)]
            accs = [jnp.where(lanemask, lax.reduce_sum(p_, (0,)), carry[0])]
            for k in range(_NEG):
                nr = k * _C + r
                q = wmv[0] * nb[nr, pl.ds(0, 16)]
                for v in range(1, _DIM // 16):
                    q = q + wmv[v] * nb[nr, pl.ds(v * 16, 16)]
                accs.append(
                    jnp.where(lanemask, lax.reduce_sum(q, (0,)), carry[1 + k]))
            flush = (r & 15) == 15

            @pl.when(flush)
            def _():
                posbuf[pl.ds(off + r - 15, 16)] = accs[0]
                for k in range(_NEG):
                    negbuf[pl.ds(k * _BPW + off + r - 15, 16)] = accs[1 + k]

            return tuple(jnp.where(flush, 0.0, a) for a in accs)

        zeros = tuple(jnp.zeros((16,), jnp.float32) for _ in range(_NEG + 1))
        lax.fori_loop(0, _C, rowstep, zeros)

    fire(0, 0)
    fire(1, 1)

    def pair(j, carry):
        ch0 = j * 2
        drain(ch0, 0)
        compute(ch0, 0)

        @pl.when(ch0 + 2 < _CH)
        def _():
            fire(ch0 + 2, 0)

        drain(ch0 + 1, 1)
        compute(ch0 + 1, 1)

        @pl.when(ch0 + 3 < _CH)
        def _():
            fire(ch0 + 3, 1)

        return carry

    lax.fori_loop(0, _CH // 2, pair, 0)

    pltpu.sync_copy(posbuf, pos_out.at[pl.ds(base, _BPW)])
    for k in range(_NEG):
        pltpu.sync_copy(negbuf.at[pl.ds(k * _BPW, _BPW)],
                        neg_out.at[pl.ds(k * _B + base, _BPW)])


_sc_call = pl.kernel(
    _sc_body,
    out_type=[
        jax.ShapeDtypeStruct((_B,), jnp.float32),
        jax.ShapeDtypeStruct((_NEG * _B,), jnp.float32),
    ],
    mesh=plsc.VectorSubcoreMesh(core_axis_name="c", subcore_axis_name="s"),
    compiler_params=pltpu.CompilerParams(needs_layout_passes=False),
    scratch_types=[
        pltpu.VMEM((_BPW,), jnp.int32),
        pltpu.VMEM((_BPW,), jnp.int32),
        pltpu.VMEM((_NEG * _BPW,), jnp.int32),
        pltpu.VMEM((_BPW * _WPB,), jnp.int32),
        pltpu.VMEM((_C, _DIM), jnp.float32),
        pltpu.VMEM((_C, _DIM), jnp.float32),
        pltpu.VMEM((_C, _DIM), jnp.float32),
        pltpu.VMEM((_C, _DIM), jnp.float32),
        pltpu.VMEM((_NPC, _DIM), jnp.float32),
        pltpu.VMEM((_NPC, _DIM), jnp.float32),
        pltpu.VMEM((_BPW,), jnp.float32),
        pltpu.VMEM((_NEG * _BPW,), jnp.float32),
        pltpu.SemaphoreType.DMA,
        pltpu.SemaphoreType.DMA,
    ],
)


def _loss_body(pos_ref, negs_ref, nm_ref, out_ref):
    pos = pos_ref[...]
    pos_l = jnp.sum(jnp.log(1.0 + jnp.exp(-jnp.clip(pos, -10.0, 10.0))))
    negs = negs_ref[...]
    nl = jnp.log(1.0 + jnp.exp(jnp.clip(negs, -10.0, 10.0))) * nm_ref[...]
    out_ref[...] = jnp.reshape(pos_l + jnp.sum(nl), (1, 1))


_loss_call = pl.pallas_call(
    _loss_body,
    out_shape=jax.ShapeDtypeStruct((1, 1), jnp.float32),
)


def kernel(data, emb0, emb1):
    widx = data[:, 1]
    cidx = data[:, 0]
    nidx = data[:, 2:2 + _NEG].T.reshape(_NEG * _B)
    nm = data[:, 2 + _NEG:].T.astype(jnp.float32)
    mbits = _dropout_mask_bits()
    pos, negs = _sc_call(widx, cidx, nidx, emb0, emb1, mbits)
    out = _loss_call(pos.reshape(_B // _DIM, _DIM), negs.reshape(_NEG, _B), nm)
    return out[0, 0]


# speedup vs baseline: 37.9896x; 1.0171x over previous
import jax
import jax.numpy as jnp
from jax import lax
from jax.experimental import pallas as pl
from jax.experimental.pallas import tpu as pltpu
from jax.experimental.pallas import tpu_sc as plsc

_VOCAB = 1000000
_DIM = 128
_NEG = 10
_DROP = 0.1
_B = 16384

_NC = 2
_NS = 16
_NW = _NC * _NS
_BPW = _B // _NW
_C = 32
_CH = _BPW // _C
_NPC = _C * _NEG
_WPB = _DIM // 32


def _dropout_mask_bits():
    with jax.ensure_compile_time_eval():
        mk = jax.random.key(12345)
        bits = (jax.random.uniform(mk, (_B, _DIM)) < (1.0 - _DROP))
        b3 = bits.reshape(_B, _WPB, 32).astype(jnp.uint32)
        words = jnp.sum(b3 << jnp.arange(32, dtype=jnp.uint32)[None, None, :],
                        axis=2, dtype=jnp.uint32)
        return lax.bitcast_convert_type(words, jnp.int32).reshape(_B * _WPB)


def _sc_body(widx, cidx, nidx, emb0, emb1, mbits, pos_out, neg_out,
             idxwv, idxcv, idxnv, bitsv, wbuf0, wbuf1, cbuf0, cbuf1,
             nbuf0, nbuf1, posbuf, negbuf, sem0, sem1):
    wid = lax.axis_index("s") * _NC + lax.axis_index("c")
    base = wid * _BPW
    lanes = lax.iota(jnp.int32, 16)
    bitsel = (jnp.full((16,), 1, jnp.int32) << lanes,
              jnp.full((16,), 1, jnp.int32) << (lanes + 16))

    pltpu.sync_copy(widx.at[pl.ds(base, _BPW)], idxwv)
    pltpu.sync_copy(cidx.at[pl.ds(base, _BPW)], idxcv)
    for k in range(_NEG):
        pltpu.sync_copy(nidx.at[pl.ds(k * _B + base, _BPW)],
                        idxnv.at[pl.ds(k * _BPW, _BPW)])
    pltpu.sync_copy(mbits.at[pl.ds(base * _WPB, _BPW * _WPB)], bitsv)

    bufs = ((wbuf0, cbuf0, nbuf0, sem0), (wbuf1, cbuf1, nbuf1, sem1))

    def _copies(ch, p):
        wb, cb, nb, sem = bufs[p]
        yield emb0.at[idxwv.at[pl.ds(ch * _C, _C)]], wb, sem
        yield emb1.at[idxcv.at[pl.ds(ch * _C, _C)]], cb, sem
        for k in range(_NEG):
            yield (emb1.at[idxnv.at[pl.ds(k * _BPW + ch * _C, _C)]],
                   nb.at[pl.ds(k * _C, _C)], sem)

    def fire(ch, p):
        for src, dst, sem in _copies(ch, p):
            pltpu.async_copy(src, dst, sem)

    def drain(ch, p):
        for src, dst, sem in _copies(ch, p):
            pltpu.make_async_copy(src, dst, sem).wait()

    def compute(ch, p):
        wb, cb, nb, _ = bufs[p]
        off = ch * _C

        def rowstep(r, carry):
            wordbase = (off + r) * _WPB
            wmv = []
            for pair in range(_WPB):
                bw = plsc.load_gather(
                    bitsv, [jnp.full((16,), wordbase + pair, jnp.int32)])
                for j in range(2):
                    v = 2 * pair + j
                    wv = wb[r, pl.ds(v * 16, 16)]
                    wmv.append(jnp.where((bw & bitsel[j]) != 0, wv, 0.0))
            lanemask = lanes == (r & 15)
            p_ = wmv[0] * cb[r, pl.ds(0, 16)]
            for v in range(1, _DIM // 16):
                p_ = p_ + wmv[v] * cb[r, pl.ds(v * 16, 16)]
            accs = [jnp.where(lanemask, lax.reduce_sum(p_, (0,)), carry[0])]
            for k in range(_NEG):
                nr = k * _C + r
                q = wmv[0] * nb[nr, pl.ds(0, 16)]
                for v in range(1, _DIM // 16):
                    q = q + wmv[v] * nb[nr, pl.ds(v * 16, 16)]
                accs.append(
                    jnp.where(lanemask, lax.reduce_sum(q, (0,)), carry[1 + k]))
            flush = (r & 15) == 15

            @pl.when(flush)
            def _():
                posbuf[pl.ds(off + r - 15, 16)] = accs[0]
                for k in range(_NEG):
                    negbuf[pl.ds(k * _BPW + off + r - 15, 16)] = accs[1 + k]

            return tuple(jnp.where(flush, 0.0, a) for a in accs)

        zeros = tuple(jnp.zeros((16,), jnp.float32) for _ in range(_NEG + 1))
        lax.fori_loop(0, _C, rowstep, zeros)

    fire(0, 0)
    fire(1, 1)

    def pair(j, carry):
        ch0 = j * 2
        drain(ch0, 0)
        compute(ch0, 0)

        @pl.when(ch0 + 2 < _CH)
        def _():
            fire(ch0 + 2, 0)

        drain(ch0 + 1, 1)
        compute(ch0 + 1, 1)

        @pl.when(ch0 + 3 < _CH)
        def _():
            fire(ch0 + 3, 1)

        return carry

    lax.fori_loop(0, _CH // 2, pair, 0)

    pltpu.sync_copy(posbuf, pos_out.at[pl.ds(base, _BPW)])
    for k in range(_NEG):
        pltpu.sync_copy(negbuf.at[pl.ds(k * _BPW, _BPW)],
                        neg_out.at[pl.ds(k * _B + base, _BPW)])


_sc_call = pl.kernel(
    _sc_body,
    out_type=[
        jax.ShapeDtypeStruct((_B,), jnp.float32),
        jax.ShapeDtypeStruct((_NEG * _B,), jnp.float32),
    ],
    mesh=plsc.VectorSubcoreMesh(core_axis_name="c", subcore_axis_name="s"),
    compiler_params=pltpu.CompilerParams(needs_layout_passes=False),
    scratch_types=[
        pltpu.VMEM((_BPW,), jnp.int32),
        pltpu.VMEM((_BPW,), jnp.int32),
        pltpu.VMEM((_NEG * _BPW,), jnp.int32),
        pltpu.VMEM((_BPW * _WPB,), jnp.int32),
        pltpu.VMEM((_C, _DIM), jnp.float32),
        pltpu.VMEM((_C, _DIM), jnp.float32),
        pltpu.VMEM((_C, _DIM), jnp.float32),
        pltpu.VMEM((_C, _DIM), jnp.float32),
        pltpu.VMEM((_NPC, _DIM), jnp.float32),
        pltpu.VMEM((_NPC, _DIM), jnp.float32),
        pltpu.VMEM((_BPW,), jnp.float32),
        pltpu.VMEM((_NEG * _BPW,), jnp.float32),
        pltpu.SemaphoreType.DMA,
        pltpu.SemaphoreType.DMA,
    ],
)


def _loss_body(pos_ref, negs_ref, nm_ref, out_ref):
    pos = pos_ref[...]
    pos_l = jnp.sum(jnp.log(1.0 + jnp.exp(-jnp.clip(pos, -10.0, 10.0))))
    negs = negs_ref[...]
    nl = jnp.log(1.0 + jnp.exp(jnp.clip(negs, -10.0, 10.0))) * nm_ref[...]
    out_ref[...] = jnp.reshape(pos_l + jnp.sum(nl), (1, 1))


_loss_call = pl.pallas_call(
    _loss_body,
    out_shape=jax.ShapeDtypeStruct((1, 1), jnp.float32),
)


def kernel(data, emb0, emb1):
    widx = data[:, 1]
    cidx = data[:, 0]
    nidx = data[:, 2:2 + _NEG].T.reshape(_NEG * _B)
    nm = data[:, 2 + _NEG:].T.reshape(_NEG * _B).astype(jnp.float32)
    mbits = _dropout_mask_bits()
    pos, negs = _sc_call(widx, cidx, nidx, emb0, emb1, mbits)
    out = _loss_call(pos, negs, nm)
    return out[0, 0]


# speedup vs baseline: 39.2498x; 1.0332x over previous
import jax
import jax.numpy as jnp
from jax import lax
from jax.experimental import pallas as pl
from jax.experimental.pallas import tpu as pltpu
from jax.experimental.pallas import tpu_sc as plsc

_VOCAB = 1000000
_DIM = 128
_NEG = 10
_DROP = 0.1
_B = 16384

_NC = 2
_NS = 16
_NW = _NC * _NS
_BPW = _B // _NW
_C = 32
_CH = _BPW // _C
_NPC = _C * _NEG
_WPB = _DIM // 32


def _dropout_mask_bits():
    with jax.ensure_compile_time_eval():
        mk = jax.random.key(12345)
        bits = (jax.random.uniform(mk, (_B, _DIM)) < (1.0 - _DROP))
        b3 = bits.reshape(_B, _WPB, 32).astype(jnp.uint32)
        words = jnp.sum(b3 << jnp.arange(32, dtype=jnp.uint32)[None, None, :],
                        axis=2, dtype=jnp.uint32)
        return lax.bitcast_convert_type(words, jnp.int32).reshape(_B * _WPB)


def _sc_body(dataT, emb0, emb1, mbits, pos_out, neg_out,
             idxwv, idxcv, idxnv, bitsv, wbuf0, wbuf1, cbuf0, cbuf1,
             nbuf0, nbuf1, posbuf, negbuf, sem0, sem1):
    wid = lax.axis_index("s") * _NC + lax.axis_index("c")
    base = wid * _BPW
    lanes = lax.iota(jnp.int32, 16)
    bitsel = (jnp.full((16,), 1, jnp.int32) << lanes,
              jnp.full((16,), 1, jnp.int32) << (lanes + 16))

    pltpu.sync_copy(dataT.at[pl.ds(_B + base, _BPW)], idxwv)
    pltpu.sync_copy(dataT.at[pl.ds(base, _BPW)], idxcv)
    for k in range(_NEG):
        pltpu.sync_copy(dataT.at[pl.ds((2 + k) * _B + base, _BPW)],
                        idxnv.at[pl.ds(k * _BPW, _BPW)])
    pltpu.sync_copy(mbits.at[pl.ds(base * _WPB, _BPW * _WPB)], bitsv)

    bufs = ((wbuf0, cbuf0, nbuf0, sem0), (wbuf1, cbuf1, nbuf1, sem1))

    def _copies(ch, p):
        wb, cb, nb, sem = bufs[p]
        yield emb0.at[idxwv.at[pl.ds(ch * _C, _C)]], wb, sem
        yield emb1.at[idxcv.at[pl.ds(ch * _C, _C)]], cb, sem
        for k in range(_NEG):
            yield (emb1.at[idxnv.at[pl.ds(k * _BPW + ch * _C, _C)]],
                   nb.at[pl.ds(k * _C, _C)], sem)

    def fire(ch, p):
        for src, dst, sem in _copies(ch, p):
            pltpu.async_copy(src, dst, sem)

    def drain(ch, p):
        for src, dst, sem in _copies(ch, p):
            pltpu.make_async_copy(src, dst, sem).wait()

    def compute(ch, p):
        wb, cb, nb, _ = bufs[p]
        off = ch * _C

        def rowstep(r, carry):
            wordbase = (off + r) * _WPB
            wmv = []
            for pair in range(_WPB):
                bw = plsc.load_gather(
                    bitsv, [jnp.full((16,), wordbase + pair, jnp.int32)])
                for j in range(2):
                    v = 2 * pair + j
                    wv = wb[r, pl.ds(v * 16, 16)]
                    wmv.append(jnp.where((bw & bitsel[j]) != 0, wv, 0.0))
            lanemask = lanes == (r & 15)
            p_ = wmv[0] * cb[r, pl.ds(0, 16)]
            for v in range(1, _DIM // 16):
                p_ = p_ + wmv[v] * cb[r, pl.ds(v * 16, 16)]
            accs = [jnp.where(lanemask, lax.reduce_sum(p_, (0,)), carry[0])]
            for k in range(_NEG):
                nr = k * _C + r
                q = wmv[0] * nb[nr, pl.ds(0, 16)]
                for v in range(1, _DIM // 16):
                    q = q + wmv[v] * nb[nr, pl.ds(v * 16, 16)]
                accs.append(
                    jnp.where(lanemask, lax.reduce_sum(q, (0,)), carry[1 + k]))
            flush = (r & 15) == 15

            @pl.when(flush)
            def _():
                posbuf[pl.ds(off + r - 15, 16)] = accs[0]
                for k in range(_NEG):
                    negbuf[pl.ds(k * _BPW + off + r - 15, 16)] = accs[1 + k]

            return tuple(jnp.where(flush, 0.0, a) for a in accs)

        zeros = tuple(jnp.zeros((16,), jnp.float32) for _ in range(_NEG + 1))
        lax.fori_loop(0, _C, rowstep, zeros)

    fire(0, 0)
    fire(1, 1)

    def pair(j, carry):
        ch0 = j * 2
        drain(ch0, 0)
        compute(ch0, 0)

        @pl.when(ch0 + 2 < _CH)
        def _():
            fire(ch0 + 2, 0)

        drain(ch0 + 1, 1)
        compute(ch0 + 1, 1)

        @pl.when(ch0 + 3 < _CH)
        def _():
            fire(ch0 + 3, 1)

        return carry

    lax.fori_loop(0, _CH // 2, pair, 0)

    pltpu.sync_copy(posbuf, pos_out.at[pl.ds(base, _BPW)])
    for k in range(_NEG):
        pltpu.sync_copy(negbuf.at[pl.ds(k * _BPW, _BPW)],
                        neg_out.at[pl.ds(k * _B + base, _BPW)])


_sc_call = pl.kernel(
    _sc_body,
    out_type=[
        jax.ShapeDtypeStruct((_B,), jnp.float32),
        jax.ShapeDtypeStruct((_NEG * _B,), jnp.float32),
    ],
    mesh=plsc.VectorSubcoreMesh(core_axis_name="c", subcore_axis_name="s"),
    compiler_params=pltpu.CompilerParams(needs_layout_passes=False),
    scratch_types=[
        pltpu.VMEM((_BPW,), jnp.int32),
        pltpu.VMEM((_BPW,), jnp.int32),
        pltpu.VMEM((_NEG * _BPW,), jnp.int32),
        pltpu.VMEM((_BPW * _WPB,), jnp.int32),
        pltpu.VMEM((_C, _DIM), jnp.float32),
        pltpu.VMEM((_C, _DIM), jnp.float32),
        pltpu.VMEM((_C, _DIM), jnp.float32),
        pltpu.VMEM((_C, _DIM), jnp.float32),
        pltpu.VMEM((_NPC, _DIM), jnp.float32),
        pltpu.VMEM((_NPC, _DIM), jnp.float32),
        pltpu.VMEM((_BPW,), jnp.float32),
        pltpu.VMEM((_NEG * _BPW,), jnp.float32),
        pltpu.SemaphoreType.DMA,
        pltpu.SemaphoreType.DMA,
    ],
)


def _loss_body(pos_ref, negs_ref, nm_ref, out_ref):
    pos = pos_ref[...]
    pos_l = jnp.sum(jnp.log(1.0 + jnp.exp(-jnp.clip(pos, -10.0, 10.0))))
    negs = negs_ref[...]
    nm = nm_ref[...].astype(jnp.float32)
    nl = jnp.log(1.0 + jnp.exp(jnp.clip(negs, -10.0, 10.0))) * nm
    out_ref[...] = jnp.reshape(pos_l + jnp.sum(nl), (1, 1))


_loss_call = pl.pallas_call(
    _loss_body,
    out_shape=jax.ShapeDtypeStruct((1, 1), jnp.float32),
)


def kernel(data, emb0, emb1):
    dataT = data.T.reshape((2 + 2 * _NEG) * _B)
    nm_i = lax.slice(dataT, ((2 + _NEG) * _B,), ((2 + 2 * _NEG) * _B,))
    mbits = _dropout_mask_bits()
    pos, negs = _sc_call(dataT, emb0, emb1, mbits)
    out = _loss_call(pos, negs, nm_i)
    return out[0, 0]
